# baseline (speedup 1.0000x reference)
import functools

import jax
import jax.numpy as jnp
from jax import lax
from jax.experimental import pallas as pl
from jax.experimental.pallas import tpu as pltpu
from jax.experimental.pallas import tpu_sc as plsc

N = 320000
D = 128
S = 10000
DA = 144

B = 2560
NBLK = N // B

NW = 32
RPW = N // NW
CS = 100
NCH = RPW // CS
STRIPE = S // 16


def _score_body(h_ref, w1_ref, b1_ref, w2_ref, b2_ref, gaug_ref):
    hb = h_ref[...]
    hidden = jnp.tanh(
        lax.dot(hb, w1_ref[...], preferred_element_type=jnp.float32)
        + b1_ref[...])
    s = lax.dot(hidden, w2_ref[...],
                preferred_element_type=jnp.float32) + b2_ref[...]
    e = jnp.exp(s)
    gaug_ref[:, :D] = hb * e
    gaug_ref[:, D:] = jnp.broadcast_to(e, (B, DA - D))


def _stage_a(h, W1, b1r, b2r, W2):
    return pl.pallas_call(
        _score_body,
        grid=(NBLK,),
        in_specs=[
            pl.BlockSpec((B, D), lambda i: (i, 0)),
            pl.BlockSpec((D, D), lambda i: (0, 0)),
            pl.BlockSpec((1, D), lambda i: (0, 0)),
            pl.BlockSpec((D, 1), lambda i: (0, 0)),
            pl.BlockSpec((1, 1), lambda i: (0, 0)),
        ],
        out_specs=pl.BlockSpec((B, DA), lambda i: (i, 0)),
        out_shape=jax.ShapeDtypeStruct((N, DA), jnp.float32),
    )(h, W1, b1r, W2, b2r)


def _sc_body(gaug_hbm, idx_hbm, zeros_hbm, npart_hbm,
             iv, gv0, gv1, num_shared, sem0, sem1):
    c = lax.axis_index("c")
    sid = lax.axis_index("s")
    w = sid * 2 + c
    base = w * RPW

    pltpu.sync_copy(zeros_hbm, num_shared.at[pl.ds(sid * STRIPE, STRIPE)])
    pltpu.sync_copy(idx_hbm.at[w], iv)
    plsc.subcore_barrier()

    def dma(jj, buf, sem):
        return pltpu.make_async_copy(
            gaug_hbm.at[pl.ds(base + jj * CS, CS)], buf, sem)

    dma(0, gv0, sem0).start()
    dma(1, gv1, sem1).start()

    @pl.loop(0, NCH, step=2)
    def _(j):
        for b, (buf, sem) in enumerate(((gv0, sem0), (gv1, sem1))):
            jj = j + b
            dma(jj, buf, sem).wait()
            pltpu.sync_copy(buf, num_shared.at[iv.at[jj]], add=True)
            nxt = jj + 2

            @pl.when(nxt < NCH)
            def _():
                dma(nxt, buf, sem).start()

    plsc.subcore_barrier()
    pltpu.sync_copy(num_shared.at[pl.ds(sid * STRIPE, STRIPE)],
                    npart_hbm.at[c, pl.ds(sid * STRIPE, STRIPE)])


def _stage_b(gaug, idx3, zeros):
    mesh = plsc.VectorSubcoreMesh(core_axis_name="c", subcore_axis_name="s")
    cp = pltpu.CompilerParams(use_tc_tiling_on_sc=False)
    f = pl.kernel(
        _sc_body,
        out_type=jax.ShapeDtypeStruct((2, S, DA), jnp.float32),
        mesh=mesh,
        scratch_types=[
            pltpu.VMEM((NCH, CS), jnp.int32),
            pltpu.VMEM((CS, DA), jnp.float32),
            pltpu.VMEM((CS, DA), jnp.float32),
            pltpu.VMEM_SHARED((S, DA), jnp.float32),
            pltpu.SemaphoreType.DMA,
            pltpu.SemaphoreType.DMA,
        ],
        compiler_params=cp,
    )
    return f(gaug, idx3, zeros)


def _combine_body(npart_ref, out_ref):
    acc = npart_ref[0] + npart_ref[1]
    num = acc[:, :D]
    den = acc[:, D:D + 1]
    out_ref[...] = num / jnp.where(den > 0.0, den, 1.0)


def _stage_c(npart):
    return pl.pallas_call(
        _combine_body,
        out_shape=jax.ShapeDtypeStruct((S, D), jnp.float32),
    )(npart)


@jax.jit
def kernel(h, batch_indices, W1, b1, W2, b2):
    b1r = b1.reshape(1, D)
    b2r = b2.reshape(1, 1)
    idx3 = batch_indices.reshape(NW, NCH, CS)
    zeros = jnp.zeros((STRIPE, DA), jnp.float32)
    gaug = _stage_a(h, W1, b1r, b2r, W2)
    npart = _stage_b(gaug, idx3, zeros)
    return _stage_c(npart)


# speedup vs baseline: 2.0497x; 2.0497x over previous
import functools

import jax
import jax.numpy as jnp
from jax import lax
from jax.experimental import pallas as pl
from jax.experimental.pallas import tpu as pltpu
from jax.experimental.pallas import tpu_sc as plsc

N = 320000
D = 128
S = 10000

B = 2560
NBLK = N // B
K = 128

NW = 32
RPW = N // NW
CS = 100
NCH = RPW // CS
STRIPE = S // 16


def _score_body(h_ref, idx_ref, w1_ref, b1_ref, w2_ref, b2_ref,
                g_ref, den_ref, dacc_ref):
    i = pl.program_id(0)

    @pl.when(i == 0)
    def _init():
        dacc_ref[...] = jnp.zeros_like(dacc_ref)

    hb = h_ref[...]
    hidden = jnp.tanh(
        lax.dot(hb, w1_ref[...], preferred_element_type=jnp.float32)
        + b1_ref[...])
    s = lax.dot(hidden, w2_ref[...],
                preferred_element_type=jnp.float32) + b2_ref[...]
    e = jnp.exp(s)
    g_ref[...] = hb * e

    idx = idx_ref[0]
    lo0 = jnp.min(idx)
    hi = jnp.max(idx)

    def cond(lo):
        return lo <= hi

    def body(lo):
        lo_c = jnp.minimum(lo - lax.rem(lo, 8), S - K)
        kio = lax.broadcasted_iota(jnp.int32, (K, B), 0)
        idxb = jnp.broadcast_to(idx, (K, B))
        oh = (idxb == kio + lo_c) & (idxb >= lo)
        ohf = oh.astype(jnp.float32)
        dwin = lax.dot(ohf, e, preferred_element_type=jnp.float32)
        dacc_ref[pl.ds(lo_c, K), :] += dwin
        return lo_c + K

    lax.while_loop(cond, body, lo0)

    @pl.when(i == NBLK - 1)
    def _finish():
        den_ref[...] = dacc_ref[...]


def _stage_a(h, idx3, W1, b1r, W2, b2r):
    return pl.pallas_call(
        _score_body,
        grid=(NBLK,),
        in_specs=[
            pl.BlockSpec((B, D), lambda i: (i, 0)),
            pl.BlockSpec((1, 1, B), lambda i: (i, 0, 0)),
            pl.BlockSpec((D, D), lambda i: (0, 0)),
            pl.BlockSpec((1, D), lambda i: (0, 0)),
            pl.BlockSpec((D, 1), lambda i: (0, 0)),
            pl.BlockSpec((1, 1), lambda i: (0, 0)),
        ],
        out_specs=[
            pl.BlockSpec((B, D), lambda i: (i, 0)),
            pl.BlockSpec((S, 1), lambda i: (0, 0)),
        ],
        out_shape=[
            jax.ShapeDtypeStruct((N, D), jnp.float32),
            jax.ShapeDtypeStruct((S, 1), jnp.float32),
        ],
        scratch_shapes=[pltpu.VMEM((S, 1), jnp.float32)],
    )(h, idx3, W1, b1r, W2, b2r)


def _sc_body(g_hbm, idx_hbm, zeros_hbm, npart_hbm,
             iv, gv0, gv1, num_shared, sem0, sem1):
    c = lax.axis_index("c")
    sid = lax.axis_index("s")
    w = sid * 2 + c
    base = w * RPW

    pltpu.sync_copy(zeros_hbm, num_shared.at[pl.ds(sid * STRIPE, STRIPE)])
    pltpu.sync_copy(idx_hbm.at[w], iv)
    plsc.subcore_barrier()

    def dma(jj, buf, sem):
        return pltpu.make_async_copy(
            g_hbm.at[pl.ds(base + jj * CS, CS)], buf, sem)

    dma(0, gv0, sem0).start()
    dma(1, gv1, sem1).start()

    @pl.loop(0, NCH, step=2)
    def _(j):
        for b, (buf, sem) in enumerate(((gv0, sem0), (gv1, sem1))):
            jj = j + b
            dma(jj, buf, sem).wait()
            pltpu.sync_copy(buf, num_shared.at[iv.at[jj]], add=True)
            nxt = jj + 2

            @pl.when(nxt < NCH)
            def _():
                dma(nxt, buf, sem).start()

    plsc.subcore_barrier()
    pltpu.sync_copy(num_shared.at[pl.ds(sid * STRIPE, STRIPE)],
                    npart_hbm.at[c, pl.ds(sid * STRIPE, STRIPE)])


def _stage_b(g, idx3, zeros):
    mesh = plsc.VectorSubcoreMesh(core_axis_name="c", subcore_axis_name="s")
    cp = pltpu.CompilerParams(use_tc_tiling_on_sc=False)
    f = pl.kernel(
        _sc_body,
        out_type=jax.ShapeDtypeStruct((2, S, D), jnp.float32),
        mesh=mesh,
        scratch_types=[
            pltpu.VMEM((NCH, CS), jnp.int32),
            pltpu.VMEM((CS, D), jnp.float32),
            pltpu.VMEM((CS, D), jnp.float32),
            pltpu.VMEM_SHARED((S, D), jnp.float32),
            pltpu.SemaphoreType.DMA,
            pltpu.SemaphoreType.DMA,
        ],
        compiler_params=cp,
    )
    return f(g, idx3, zeros)


def _combine_body(npart_ref, den_ref, out_ref):
    num = npart_ref[0] + npart_ref[1]
    den = den_ref[...]
    out_ref[...] = num / jnp.where(den > 0.0, den, 1.0)


def _stage_c(npart, den):
    return pl.pallas_call(
        _combine_body,
        out_shape=jax.ShapeDtypeStruct((S, D), jnp.float32),
    )(npart, den)


@jax.jit
def kernel(h, batch_indices, W1, b1, W2, b2):
    b1r = b1.reshape(1, D)
    b2r = b2.reshape(1, 1)
    idx3a = batch_indices.reshape(NBLK, 1, B)
    idx3b = batch_indices.reshape(NW, NCH, CS)
    zeros = jnp.zeros((STRIPE, D), jnp.float32)
    g, den = _stage_a(h, idx3a, W1, b1r, W2, b2r)
    npart = _stage_b(g, idx3b, zeros)
    return _stage_c(npart, den)
